# baseline (speedup 1.0000x reference)
import functools
import math

import jax
import jax.numpy as jnp
from jax import lax
from jax.experimental import pallas as pl
from jax.experimental.pallas import tpu as pltpu
from jax.experimental.pallas import tpu_sc as plsc

N_NODES = 10000
N_EDGES = 320000
D = 128
H = 4
DH = D // H
N_RADIAL = 16
HIDDEN = 128

EDGE_BLOCK = 1000
PIPE_CHUNKS = 1
NODE_BLOCK = 1024
N_PAD = 10240

NC = 2
NS = 16
NW = NC * NS
LANES = 16


def _head_expand_mat(dtype=jnp.float32):
  col = lax.broadcasted_iota(jnp.int32, (H, D), 1) // DH
  row = lax.broadcasted_iota(jnp.int32, (H, D), 0)
  return (col == row).astype(dtype)


def _edge_body(esa_ref, xs_ref, xd_ref, ea_ref, cut_ref,
               wk1_ref, wk2_ref, wk3_ref, wv1_ref, wv2_ref, wv3_ref,
               wdot_ref, exp_ref, u_ref):
  f32 = jnp.float32
  bf16 = jnp.bfloat16
  s_in = 1.0 / math.sqrt(N_RADIAL)
  s_h = 1.0 / math.sqrt(HIDDEN)

  esa = esa_ref[...].astype(bf16)
  hk = jax.nn.gelu(jnp.dot(esa, wk1_ref[...], preferred_element_type=f32) * s_in)
  hk = jnp.dot(hk.astype(bf16), wk2_ref[...], preferred_element_type=f32)
  hk = jax.nn.gelu(hk * s_h)
  wk = jnp.dot(hk.astype(bf16), wk3_ref[...], preferred_element_type=f32) * s_h
  hv = jax.nn.gelu(jnp.dot(esa, wv1_ref[...], preferred_element_type=f32) * s_in)
  hv = jnp.dot(hv.astype(bf16), wv2_ref[...], preferred_element_type=f32)
  hv = jax.nn.gelu(hv * s_h)
  wv = jnp.dot(hv.astype(bf16), wv3_ref[...], preferred_element_type=f32) * s_h

  xs = xs_ref[...].astype(f32)
  ea = ea_ref[...]
  ek = wk * xs * ea
  m = jnp.dot(ek.astype(bf16), wdot_ref[...], preferred_element_type=f32)
  xd = xd_ref[...].astype(f32)
  dots = []
  for w in range(H):
    dots.append(jnp.sum(m[:, w * D:(w + 1) * D] * xd, axis=1, keepdims=True))
  dot = jnp.concatenate(dots, axis=1) * (1.0 / D)

  cut = cut_ref[...]
  edot2 = jnp.exp(0.5 * dot)
  exp_ref[...] = cut * edot2 * edot2
  sexp = jnp.sqrt(cut) * edot2

  ev = wv * xs * ea
  srep = jnp.dot(sexp, _head_expand_mat(), preferred_element_type=f32)
  u_ref[...] = ev * srep


def _tc_edge(esa, xs, xd, ea, cut, Wk1, Wk2, Wk3, Wv1, Wv2, Wv3, Wdot_r,
             interpret=False):
  B = EDGE_BLOCK
  ne = esa.shape[0]
  grid = (ne // B,)
  def eb(j): return pl.BlockSpec((B, j), lambda i: (i, 0))
  def full(a): return pl.BlockSpec(a.shape, lambda i: (0,) * a.ndim)
  return pl.pallas_call(
      _edge_body,
      grid=grid,
      in_specs=[eb(N_RADIAL), eb(D), eb(D), eb(1), eb(1),
                full(Wk1), full(Wk2), full(Wk3),
                full(Wv1), full(Wv2), full(Wv3), full(Wdot_r)],
      out_specs=[eb(H), eb(D)],
      out_shape=[jax.ShapeDtypeStruct((ne, H), jnp.float32),
                 jax.ShapeDtypeStruct((ne, D), jnp.float32)],
      interpret=interpret,
  )(esa, xs, xd, ea, cut, Wk1, Wk2, Wk3, Wv1, Wv2, Wv3, Wdot_r)


def _final_body(acc_ref, zp_ref, wlin_ref, out_ref):
  nacc = acc_ref.shape[0]
  npart = zp_ref.shape[0]
  acc = acc_ref[0]
  for p in range(1, nacc):
    acc = acc + acc_ref[p]
  z = zp_ref[0]
  for p in range(1, npart):
    z = z + zp_ref[p]
  z = jnp.where(z == 0.0, 1.0, z)
  rs = lax.rsqrt(z)
  rsrep = lax.dot_general(rs, _head_expand_mat(),
                          (((0,), (0,)), ((), ())),
                          preferred_element_type=jnp.float32)
  y = acc * rsrep
  out_ref[...] = jnp.dot(y, wlin_ref[...],
                         preferred_element_type=jnp.float32) * (1.0 / math.sqrt(D))


def _tc_final(acc, zpart, Wlin, interpret=False):
  B = NODE_BLOCK
  grid = (N_PAD // B,)
  na, np_ = acc.shape[0], zpart.shape[0]
  return pl.pallas_call(
      _final_body,
      grid=grid,
      in_specs=[pl.BlockSpec((na, B, D), lambda i: (0, i, 0)),
                pl.BlockSpec((np_, H, B), lambda i: (0, 0, i)),
                pl.BlockSpec((D, D), lambda i: (0, 0))],
      out_specs=pl.BlockSpec((B, D), lambda i: (i, 0)),
      out_shape=jax.ShapeDtypeStruct((N_PAD, D), jnp.float32),
      interpret=interpret,
  )(acc, zpart, Wlin)


_GK = 1000


def _sc_gather(node_f, edge_src, edge_dst):
  ne = edge_src.shape[0]
  epw = ne // NW
  mesh = plsc.VectorSubcoreMesh(core_axis_name="c", subcore_axis_name="s")

  @functools.partial(
      pl.kernel,
      out_type=[jax.ShapeDtypeStruct((ne, D), jnp.float32),
                jax.ShapeDtypeStruct((ne, D), jnp.float32)],
      mesh=mesh,
      compiler_params=pltpu.CompilerParams(needs_layout_passes=False),
      scratch_types=[pltpu.VMEM((_GK,), jnp.int32),
                     pltpu.VMEM((_GK, D), jnp.float32),
                     pltpu.SemaphoreType.DMA],
  )
  def body(node_hbm, src_hbm, dst_hbm, xs_hbm, xd_hbm, idx_v, rows_v, sem):
    wid = lax.axis_index("s") * NC + lax.axis_index("c")

    def chunk(base, idx_hbm, out_hbm):
      pltpu.sync_copy(idx_hbm.at[pl.ds(base, _GK)], idx_v)
      pltpu.async_copy(node_hbm.at[idx_v], rows_v, sem).wait()
      pltpu.sync_copy(rows_v, out_hbm.at[pl.ds(base, _GK)])

    def loop_body(c, carry):
      base = wid * epw + c * _GK
      chunk(base, src_hbm, xs_hbm)
      chunk(base, dst_hbm, xd_hbm)
      return carry

    lax.fori_loop(0, epw // _GK, loop_body, 0)

  return body(node_f, edge_src, edge_dst)


def _sc_zscatter(expv_flat, edge_dst):
  epw = edge_dst.shape[0] // NW
  mesh = plsc.VectorSubcoreMesh(core_axis_name="c", subcore_axis_name="s")
  ZW = N_NODES * H

  @functools.partial(
      pl.kernel,
      out_type=jax.ShapeDtypeStruct((NW, H, N_PAD), jnp.float32),
      mesh=mesh,
      compiler_params=pltpu.CompilerParams(needs_layout_passes=False),
      scratch_types=[pltpu.VMEM((epw * H,), jnp.float32),
                     pltpu.VMEM((epw,), jnp.int32),
                     pltpu.VMEM((H, N_PAD), jnp.float32)],
  )
  def body(exp_hbm, dst_hbm, zp_hbm, exp_v, dst_v, z_v):
    wid = lax.axis_index("s") * NC + lax.axis_index("c")
    pltpu.sync_copy(exp_hbm.at[pl.ds(wid * epw * H, epw * H)], exp_v)
    pltpu.sync_copy(dst_hbm.at[pl.ds(wid * epw, epw)], dst_v)

    zero = jnp.zeros((LANES,), jnp.float32)

    def zbody(i, carry):
      h = i // (N_PAD // LANES)
      n = i % (N_PAD // LANES)
      z_v[h, pl.ds(n * LANES, LANES)] = zero
      return carry

    lax.fori_loop(0, H * N_PAD // LANES, zbody, 0)

    lane = lax.iota(jnp.int32, LANES)
    lane_e = lane >> 2
    lane_h = lane & (H - 1)

    def sbody(g, carry):
      dstg = plsc.load_gather(dst_v, [g * 4 + lane_e])
      vals = exp_v[pl.ds(g * LANES, LANES)]
      plsc.addupdate_scatter(z_v, [lane_h, dstg], vals)
      return carry

    lax.fori_loop(0, epw * H // LANES, sbody, 0)
    pltpu.sync_copy(z_v, zp_hbm.at[wid])

  return body(expv_flat, edge_dst)


_UK = 200
_DRAIN = 80
_NDCHUNK = N_PAD // _DRAIN


def _sc_uscatter(u, edge_dst):
  epw = edge_dst.shape[0] // NW
  mesh = plsc.VectorSubcoreMesh(core_axis_name="c", subcore_axis_name="s")

  @functools.partial(
      pl.kernel,
      out_type=jax.ShapeDtypeStruct((NC, N_PAD, D), jnp.float32),
      mesh=mesh,
      compiler_params=pltpu.CompilerParams(needs_layout_passes=False),
      scratch_types=[pltpu.VMEM((_UK, D), jnp.float32),
                     pltpu.VMEM((_UK,), jnp.int32),
                     pltpu.VMEM_SHARED((N_PAD, D), jnp.float32)],
  )
  def body(u_hbm, dst_hbm, out_hbm, u_v, dst_v, acc_sh):
    cid = lax.axis_index("c")
    sid = lax.axis_index("s")
    wid = sid * NC + cid

    zero = jnp.zeros((LANES,), jnp.float32)

    def zfill(i, carry):
      u_v[i // (D // LANES), pl.ds((i % (D // LANES)) * LANES, LANES)] = zero
      return carry

    lax.fori_loop(0, _DRAIN * D // LANES, zfill, 0)

    def zcopy(j, carry):
      c = sid + j * NS

      @pl.when(c < _NDCHUNK)
      def _():
        pltpu.sync_copy(u_v.at[pl.ds(0, _DRAIN)], acc_sh.at[pl.ds(c * _DRAIN, _DRAIN)])

      return carry

    lax.fori_loop(0, (_NDCHUNK + NS - 1) // NS, zcopy, 0)
    plsc.subcore_barrier()

    def cbody(k, carry):
      base = wid * epw + k * _UK
      pltpu.sync_copy(dst_hbm.at[pl.ds(base, _UK)], dst_v)
      pltpu.sync_copy(u_hbm.at[pl.ds(base, _UK)], u_v)
      pltpu.sync_copy(u_v, acc_sh.at[dst_v], add=True)
      return carry

    lax.fori_loop(0, epw // _UK, cbody, 0)
    plsc.subcore_barrier()

    def dbody(j, carry):
      c = sid + j * NS

      @pl.when(c < _NDCHUNK)
      def _():
        row = c * _DRAIN
        dr_v = u_v.at[pl.ds(0, _DRAIN)]
        pltpu.sync_copy(acc_sh.at[pl.ds(row, _DRAIN)], dr_v)
        pltpu.sync_copy(dr_v, out_hbm.at[cid, pl.ds(row, _DRAIN)])

      return carry

    lax.fori_loop(0, (_NDCHUNK + NS - 1) // NS, dbody, 0)

  return body(u, edge_dst)


def kernel(edge_src, edge_dst, edge_scalar_attr, edge_attr, edge_weight_cutoff,
           node_f, Wk1, Wk2, Wk3, Wv1, Wv2, Wv3, Wdot, Wlin):
  bf16 = jnp.bfloat16
  Wdot_r = jnp.transpose(Wdot, (1, 2, 0)).reshape(D, H * D).astype(bf16)
  Wk1, Wk2, Wk3 = Wk1.astype(bf16), Wk2.astype(bf16), Wk3.astype(bf16)
  Wv1, Wv2, Wv3 = Wv1.astype(bf16), Wv2.astype(bf16), Wv3.astype(bf16)

  ea = edge_attr
  cut = edge_weight_cutoff[:, None]

  nch = PIPE_CHUNKS
  ce = N_EDGES // nch

  gathered = [_sc_gather(node_f,
                         lax.slice_in_dim(edge_src, c * ce, (c + 1) * ce),
                         lax.slice_in_dim(edge_dst, c * ce, (c + 1) * ce))
              for c in range(nch)]
  exps, us, accs = [], [], []
  for c in range(nch):
    xs, xd = gathered[c]
    expv, u = _tc_edge(lax.slice_in_dim(edge_scalar_attr, c * ce, (c + 1) * ce),
                       xs, xd,
                       lax.slice_in_dim(ea, c * ce, (c + 1) * ce),
                       lax.slice_in_dim(cut, c * ce, (c + 1) * ce),
                       Wk1, Wk2, Wk3, Wv1, Wv2, Wv3, Wdot_r)
    exps.append(expv)
    us.append(u)
  for c in range(nch):
    dst_c = lax.slice_in_dim(edge_dst, c * ce, (c + 1) * ce)
    accs.append(_sc_uscatter(us[c], dst_c))
  zpart = _sc_zscatter(jnp.concatenate(exps).reshape(-1), edge_dst)
  acc = accs[0] if nch == 1 else jnp.concatenate(accs, axis=0)

  return lax.slice_in_dim(_tc_final(acc, zpart, Wlin), 0, N_NODES)


# speedup vs baseline: 1.2573x; 1.2573x over previous
import functools
import math

import jax
import jax.numpy as jnp
from jax import lax
from jax.experimental import pallas as pl
from jax.experimental.pallas import tpu as pltpu
from jax.experimental.pallas import tpu_sc as plsc

N_NODES = 10000
N_EDGES = 320000
D = 128
H = 4
DH = D // H
N_RADIAL = 16
HIDDEN = 128

EDGE_BLOCK = 1280
PIPE_CHUNKS = 1
NODE_BLOCK = 1024
N_PAD = 10240

NC = 2
NS = 16
NW = NC * NS
LANES = 16


def _head_expand_mat(dtype=jnp.float32):
  col = lax.broadcasted_iota(jnp.int32, (H, D), 1) // DH
  row = lax.broadcasted_iota(jnp.int32, (H, D), 0)
  return (col == row).astype(dtype)


def _edge_body(esa_ref, xs_ref, xd_ref, ea_ref, cut_ref,
               wk1_ref, wk2_ref, wk3_ref, wv1_ref, wv2_ref, wv3_ref,
               wdot_ref, exp_ref, u_ref):
  f32 = jnp.float32
  bf16 = jnp.bfloat16
  s_in = 1.0 / math.sqrt(N_RADIAL)
  s_h = 1.0 / math.sqrt(HIDDEN)
  dn0 = (((0,), (0,)), ((), ()))

  esa = esa_ref[...].astype(bf16)
  hk = jax.nn.gelu(
      lax.dot_general(esa, wk1_ref[...], dn0, preferred_element_type=f32) * s_in)
  hk = jnp.dot(hk.astype(bf16), wk2_ref[...], preferred_element_type=f32)
  hk = jax.nn.gelu(hk * s_h)
  wk = jnp.dot(hk.astype(bf16), wk3_ref[...], preferred_element_type=f32) * s_h
  hv = jax.nn.gelu(
      lax.dot_general(esa, wv1_ref[...], dn0, preferred_element_type=f32) * s_in)
  hv = jnp.dot(hv.astype(bf16), wv2_ref[...], preferred_element_type=f32)
  hv = jax.nn.gelu(hv * s_h)
  wv = jnp.dot(hv.astype(bf16), wv3_ref[...], preferred_element_type=f32) * s_h

  xs = xs_ref[...].astype(f32)
  ea = lax.transpose(ea_ref[...], (1, 0))
  ek = wk * xs * ea
  m = jnp.dot(ek.astype(bf16), wdot_ref[...], preferred_element_type=f32)
  xd = xd_ref[...].astype(f32)
  dots = []
  for w in range(H):
    dots.append(jnp.sum(m[:, w * D:(w + 1) * D] * xd, axis=1, keepdims=True))
  dot = jnp.concatenate(dots, axis=1) * (1.0 / D)

  cut = lax.transpose(cut_ref[...], (1, 0))
  edot2 = jnp.exp(0.5 * dot)
  exp_ref[...] = lax.transpose(cut * edot2 * edot2, (1, 0))
  sexp = jnp.sqrt(cut) * edot2

  ev = wv * xs * ea
  srep = jnp.dot(sexp, _head_expand_mat(), preferred_element_type=f32)
  u_ref[...] = ev * srep


def _tc_edge(esaT, xs, xd, eaT, cutT, Wk1, Wk2, Wk3, Wv1, Wv2, Wv3, Wdot_r,
             interpret=False):
  B = EDGE_BLOCK
  ne = esaT.shape[1]
  grid = (ne // B,)
  def eb(j): return pl.BlockSpec((B, j), lambda i: (i, 0))
  def ebT(j): return pl.BlockSpec((j, B), lambda i: (0, i))
  def full(a): return pl.BlockSpec(a.shape, lambda i: (0,) * a.ndim)
  return pl.pallas_call(
      _edge_body,
      grid=grid,
      in_specs=[ebT(N_RADIAL), eb(D), eb(D), ebT(1), ebT(1),
                full(Wk1), full(Wk2), full(Wk3),
                full(Wv1), full(Wv2), full(Wv3), full(Wdot_r)],
      out_specs=[ebT(H), eb(D)],
      out_shape=[jax.ShapeDtypeStruct((H, ne), jnp.float32),
                 jax.ShapeDtypeStruct((ne, D), jnp.float32)],
      interpret=interpret,
  )(esaT, xs, xd, eaT, cutT, Wk1, Wk2, Wk3, Wv1, Wv2, Wv3, Wdot_r)


def _final_body(acc_ref, zp_ref, wlin_ref, out_ref):
  nacc = acc_ref.shape[0]
  npart = zp_ref.shape[0]
  acc = acc_ref[0]
  for p in range(1, nacc):
    acc = acc + acc_ref[p]
  z = zp_ref[0]
  for p in range(1, npart):
    z = z + zp_ref[p]
  z = jnp.where(z == 0.0, 1.0, z)
  rs = lax.rsqrt(z)
  rsrep = lax.dot_general(rs, _head_expand_mat(),
                          (((0,), (0,)), ((), ())),
                          preferred_element_type=jnp.float32)
  y = acc * rsrep
  out_ref[...] = jnp.dot(y, wlin_ref[...],
                         preferred_element_type=jnp.float32) * (1.0 / math.sqrt(D))


def _tc_final(acc, zpart, Wlin, interpret=False):
  B = NODE_BLOCK
  grid = (N_PAD // B,)
  na, np_ = acc.shape[0], zpart.shape[0]
  return pl.pallas_call(
      _final_body,
      grid=grid,
      in_specs=[pl.BlockSpec((na, B, D), lambda i: (0, i, 0)),
                pl.BlockSpec((np_, H, B), lambda i: (0, 0, i)),
                pl.BlockSpec((D, D), lambda i: (0, 0))],
      out_specs=pl.BlockSpec((B, D), lambda i: (i, 0)),
      out_shape=jax.ShapeDtypeStruct((N_PAD, D), jnp.float32),
      interpret=interpret,
  )(acc, zpart, Wlin)


_GK = 1000


def _sc_gather(node_f, edge_src, edge_dst):
  ne = edge_src.shape[0]
  epw = ne // NW
  mesh = plsc.VectorSubcoreMesh(core_axis_name="c", subcore_axis_name="s")

  @functools.partial(
      pl.kernel,
      out_type=[jax.ShapeDtypeStruct((ne, D), jnp.float32),
                jax.ShapeDtypeStruct((ne, D), jnp.float32)],
      mesh=mesh,
      compiler_params=pltpu.CompilerParams(needs_layout_passes=False),
      scratch_types=[pltpu.VMEM((_GK,), jnp.int32),
                     pltpu.VMEM((_GK, D), jnp.float32),
                     pltpu.SemaphoreType.DMA],
  )
  def body(node_hbm, src_hbm, dst_hbm, xs_hbm, xd_hbm, idx_v, rows_v, sem):
    wid = lax.axis_index("s") * NC + lax.axis_index("c")

    def chunk(base, idx_hbm, out_hbm):
      pltpu.sync_copy(idx_hbm.at[pl.ds(base, _GK)], idx_v)
      pltpu.async_copy(node_hbm.at[idx_v], rows_v, sem).wait()
      pltpu.sync_copy(rows_v, out_hbm.at[pl.ds(base, _GK)])

    def loop_body(c, carry):
      base = wid * epw + c * _GK
      chunk(base, src_hbm, xs_hbm)
      chunk(base, dst_hbm, xd_hbm)
      return carry

    lax.fori_loop(0, epw // _GK, loop_body, 0)

  return body(node_f, edge_src, edge_dst)


def _sc_zscatter(expT, edge_dst):
  ne = edge_dst.shape[0]
  epw = ne // NW
  W = 10240
  mesh = plsc.VectorSubcoreMesh(core_axis_name="c", subcore_axis_name="s")

  @functools.partial(
      pl.kernel,
      out_type=jax.ShapeDtypeStruct((NW, H, N_PAD), jnp.float32),
      mesh=mesh,
      compiler_params=pltpu.CompilerParams(needs_layout_passes=False),
      scratch_types=[pltpu.VMEM((H, W), jnp.float32),
                     pltpu.VMEM((epw,), jnp.int32),
                     pltpu.VMEM((H, N_PAD), jnp.float32)],
  )
  def body(exp_hbm, dst_hbm, zp_hbm, exp_v, dst_v, z_v):
    wid = lax.axis_index("s") * NC + lax.axis_index("c")
    start = wid * epw
    base = jnp.minimum((start // 128) * 128, ne - W)
    off = start - base
    pltpu.sync_copy(exp_hbm.at[:, pl.ds(base, W)], exp_v)
    pltpu.sync_copy(dst_hbm.at[pl.ds(start, epw)], dst_v)

    zero = jnp.zeros((LANES,), jnp.float32)

    def zbody(i, carry):
      h = i // (N_PAD // LANES)
      n = i % (N_PAD // LANES)
      z_v[h, pl.ds(n * LANES, LANES)] = zero
      return carry

    lax.fori_loop(0, H * N_PAD // LANES, zbody, 0)

    def sbody(g, carry):
      dstg = dst_v[pl.ds(g * LANES, LANES)]
      for h in range(H):
        vals = exp_v[h, pl.ds(off + g * LANES, LANES)]
        hvec = jnp.full((LANES,), h, jnp.int32)
        plsc.addupdate_scatter(z_v, [hvec, dstg], vals)
      return carry

    lax.fori_loop(0, epw // LANES, sbody, 0)
    pltpu.sync_copy(z_v, zp_hbm.at[wid])

  return body(expT, edge_dst)


_UK = 200
_DRAIN = 80
_NDCHUNK = N_PAD // _DRAIN


def _sc_uscatter(u, edge_dst):
  epw = edge_dst.shape[0] // NW
  mesh = plsc.VectorSubcoreMesh(core_axis_name="c", subcore_axis_name="s")

  @functools.partial(
      pl.kernel,
      out_type=jax.ShapeDtypeStruct((NC, N_PAD, D), jnp.float32),
      mesh=mesh,
      compiler_params=pltpu.CompilerParams(needs_layout_passes=False),
      scratch_types=[pltpu.VMEM((_UK, D), jnp.float32),
                     pltpu.VMEM((_UK,), jnp.int32),
                     pltpu.VMEM_SHARED((N_PAD, D), jnp.float32)],
  )
  def body(u_hbm, dst_hbm, out_hbm, u_v, dst_v, acc_sh):
    cid = lax.axis_index("c")
    sid = lax.axis_index("s")
    wid = sid * NC + cid

    zero = jnp.zeros((LANES,), jnp.float32)

    def zfill(i, carry):
      u_v[i // (D // LANES), pl.ds((i % (D // LANES)) * LANES, LANES)] = zero
      return carry

    lax.fori_loop(0, _DRAIN * D // LANES, zfill, 0)

    def zcopy(j, carry):
      c = sid + j * NS

      @pl.when(c < _NDCHUNK)
      def _():
        pltpu.sync_copy(u_v.at[pl.ds(0, _DRAIN)], acc_sh.at[pl.ds(c * _DRAIN, _DRAIN)])

      return carry

    lax.fori_loop(0, (_NDCHUNK + NS - 1) // NS, zcopy, 0)
    plsc.subcore_barrier()

    def cbody(k, carry):
      base = wid * epw + k * _UK
      pltpu.sync_copy(dst_hbm.at[pl.ds(base, _UK)], dst_v)
      pltpu.sync_copy(u_hbm.at[pl.ds(base, _UK)], u_v)
      pltpu.sync_copy(u_v, acc_sh.at[dst_v], add=True)
      return carry

    lax.fori_loop(0, epw // _UK, cbody, 0)
    plsc.subcore_barrier()

    def dbody(j, carry):
      c = sid + j * NS

      @pl.when(c < _NDCHUNK)
      def _():
        row = c * _DRAIN
        dr_v = u_v.at[pl.ds(0, _DRAIN)]
        pltpu.sync_copy(acc_sh.at[pl.ds(row, _DRAIN)], dr_v)
        pltpu.sync_copy(dr_v, out_hbm.at[cid, pl.ds(row, _DRAIN)])

      return carry

    lax.fori_loop(0, (_NDCHUNK + NS - 1) // NS, dbody, 0)

  return body(u, edge_dst)


def kernel(edge_src, edge_dst, edge_scalar_attr, edge_attr, edge_weight_cutoff,
           node_f, Wk1, Wk2, Wk3, Wv1, Wv2, Wv3, Wdot, Wlin):
  bf16 = jnp.bfloat16
  Wdot_r = jnp.transpose(Wdot, (1, 2, 0)).reshape(D, H * D).astype(bf16)
  Wk1, Wk2, Wk3 = Wk1.astype(bf16), Wk2.astype(bf16), Wk3.astype(bf16)
  Wv1, Wv2, Wv3 = Wv1.astype(bf16), Wv2.astype(bf16), Wv3.astype(bf16)

  esaT = edge_scalar_attr.T
  eaT = edge_attr.T
  cutT = edge_weight_cutoff[None, :]

  nch = PIPE_CHUNKS
  ce = N_EDGES // nch

  gathered = [_sc_gather(node_f,
                         lax.slice_in_dim(edge_src, c * ce, (c + 1) * ce),
                         lax.slice_in_dim(edge_dst, c * ce, (c + 1) * ce))
              for c in range(nch)]
  exps, us, accs = [], [], []
  for c in range(nch):
    xs, xd = gathered[c]
    expv, u = _tc_edge(lax.slice_in_dim(esaT, c * ce, (c + 1) * ce, axis=1),
                       xs, xd,
                       lax.slice_in_dim(eaT, c * ce, (c + 1) * ce, axis=1),
                       lax.slice_in_dim(cutT, c * ce, (c + 1) * ce, axis=1),
                       Wk1, Wk2, Wk3, Wv1, Wv2, Wv3, Wdot_r)
    exps.append(expv)
    us.append(u)
  for c in range(nch):
    dst_c = lax.slice_in_dim(edge_dst, c * ce, (c + 1) * ce)
    accs.append(_sc_uscatter(us[c], dst_c))
  expT = exps[0] if nch == 1 else jnp.concatenate(exps, axis=1)
  zpart = _sc_zscatter(expT, edge_dst)
  acc = accs[0] if nch == 1 else jnp.concatenate(accs, axis=0)

  return lax.slice_in_dim(_tc_final(acc, zpart, Wlin), 0, N_NODES)


# speedup vs baseline: 1.6129x; 1.2828x over previous
import functools
import math

import jax
import jax.numpy as jnp
from jax import lax
from jax.experimental import pallas as pl
from jax.experimental.pallas import tpu as pltpu
from jax.experimental.pallas import tpu_sc as plsc

N_NODES = 10000
N_EDGES = 320000
D = 128
H = 4
DH = D // H
N_RADIAL = 16
HIDDEN = 128

EDGE_BLOCK = 1280
PIPE_CHUNKS = 2
NODE_BLOCK = 1024
N_PAD = 10240

NC = 2
NS = 16
NW = NC * NS
LANES = 16


def _head_expand_mat(dtype=jnp.float32):
  col = lax.broadcasted_iota(jnp.int32, (H, D), 1) // DH
  row = lax.broadcasted_iota(jnp.int32, (H, D), 0)
  return (col == row).astype(dtype)


def _edge_body(esa_ref, xs_ref, xd_ref, ea_ref, cut_ref,
               wk1_ref, wk2_ref, wk3_ref, wv1_ref, wv2_ref, wv3_ref,
               wdot_ref, exp_ref, u_ref):
  f32 = jnp.float32
  bf16 = jnp.bfloat16
  s_in = 1.0 / math.sqrt(N_RADIAL)
  s_h = 1.0 / math.sqrt(HIDDEN)
  dn0 = (((0,), (0,)), ((), ()))

  esa = esa_ref[...].astype(bf16)
  hk = jax.nn.gelu(
      lax.dot_general(esa, wk1_ref[...], dn0, preferred_element_type=f32) * s_in)
  hk = jnp.dot(hk.astype(bf16), wk2_ref[...], preferred_element_type=f32)
  hk = jax.nn.gelu(hk * s_h)
  wk = jnp.dot(hk.astype(bf16), wk3_ref[...], preferred_element_type=f32) * s_h
  hv = jax.nn.gelu(
      lax.dot_general(esa, wv1_ref[...], dn0, preferred_element_type=f32) * s_in)
  hv = jnp.dot(hv.astype(bf16), wv2_ref[...], preferred_element_type=f32)
  hv = jax.nn.gelu(hv * s_h)
  wv = jnp.dot(hv.astype(bf16), wv3_ref[...], preferred_element_type=f32) * s_h

  xs = xs_ref[...].astype(f32)
  ea = lax.transpose(ea_ref[...], (1, 0))
  ek = wk * xs * ea
  m = jnp.dot(ek.astype(bf16), wdot_ref[...], preferred_element_type=f32)
  xd = xd_ref[...].astype(f32)
  xd4 = jnp.concatenate([xd, xd, xd, xd], axis=1)
  prod = m * xd4
  selc = lax.broadcasted_iota(jnp.int32, (H * D, H), 0) // D
  selr = lax.broadcasted_iota(jnp.int32, (H * D, H), 1)
  sel = (selc == selr).astype(f32)
  dot = jnp.dot(prod, sel, preferred_element_type=f32) * (1.0 / D)

  cut = lax.transpose(cut_ref[...], (1, 0))
  edot2 = jnp.exp(0.5 * dot)
  exp_ref[...] = lax.transpose(cut * edot2 * edot2, (1, 0))
  sexp = jnp.sqrt(cut) * edot2

  ev = wv * xs * ea
  srep = jnp.dot(sexp, _head_expand_mat(), preferred_element_type=f32)
  u_ref[...] = ev * srep


def _tc_edge(esaT, xs, xd, eaT, cutT, Wk1, Wk2, Wk3, Wv1, Wv2, Wv3, Wdot_r,
             interpret=False):
  B = EDGE_BLOCK
  ne = esaT.shape[1]
  grid = (ne // B,)
  def eb(j): return pl.BlockSpec((B, j), lambda i: (i, 0))
  def ebT(j): return pl.BlockSpec((j, B), lambda i: (0, i))
  def full(a): return pl.BlockSpec(a.shape, lambda i: (0,) * a.ndim)
  return pl.pallas_call(
      _edge_body,
      grid=grid,
      in_specs=[ebT(N_RADIAL), eb(D), eb(D), ebT(1), ebT(1),
                full(Wk1), full(Wk2), full(Wk3),
                full(Wv1), full(Wv2), full(Wv3), full(Wdot_r)],
      out_specs=[ebT(H), eb(D)],
      out_shape=[jax.ShapeDtypeStruct((H, ne), jnp.float32),
                 jax.ShapeDtypeStruct((ne, D), jnp.float32)],
      interpret=interpret,
  )(esaT, xs, xd, eaT, cutT, Wk1, Wk2, Wk3, Wv1, Wv2, Wv3, Wdot_r)


def _final_body(acc_ref, zp_ref, wlin_ref, out_ref):
  nacc = acc_ref.shape[0]
  npart = zp_ref.shape[0]
  acc = acc_ref[0]
  for p in range(1, nacc):
    acc = acc + acc_ref[p]
  z = zp_ref[0]
  for p in range(1, npart):
    z = z + zp_ref[p]
  z = jnp.where(z == 0.0, 1.0, z)
  rs = lax.rsqrt(z)
  rsrep = lax.dot_general(rs, _head_expand_mat(),
                          (((0,), (0,)), ((), ())),
                          preferred_element_type=jnp.float32)
  y = acc * rsrep
  out_ref[...] = jnp.dot(y, wlin_ref[...],
                         preferred_element_type=jnp.float32) * (1.0 / math.sqrt(D))


def _tc_final(acc, zpart, Wlin, interpret=False):
  B = NODE_BLOCK
  grid = (N_PAD // B,)
  na, np_ = acc.shape[0], zpart.shape[0]
  return pl.pallas_call(
      _final_body,
      grid=grid,
      in_specs=[pl.BlockSpec((na, B, D), lambda i: (0, i, 0)),
                pl.BlockSpec((np_, H, B), lambda i: (0, 0, i)),
                pl.BlockSpec((D, D), lambda i: (0, 0))],
      out_specs=pl.BlockSpec((B, D), lambda i: (i, 0)),
      out_shape=jax.ShapeDtypeStruct((N_PAD, D), jnp.float32),
      interpret=interpret,
  )(acc, zpart, Wlin)


_GK = 1000


def _sc_gather(node_f, edge_src, edge_dst):
  ne = edge_src.shape[0]
  epw = ne // NW
  mesh = plsc.VectorSubcoreMesh(core_axis_name="c", subcore_axis_name="s")

  @functools.partial(
      pl.kernel,
      out_type=[jax.ShapeDtypeStruct((ne, D), jnp.float32),
                jax.ShapeDtypeStruct((ne, D), jnp.float32)],
      mesh=mesh,
      compiler_params=pltpu.CompilerParams(needs_layout_passes=False),
      scratch_types=[pltpu.VMEM((_GK,), jnp.int32),
                     pltpu.VMEM((_GK, D), jnp.float32),
                     pltpu.SemaphoreType.DMA],
  )
  def body(node_hbm, src_hbm, dst_hbm, xs_hbm, xd_hbm, idx_v, rows_v, sem):
    wid = lax.axis_index("s") * NC + lax.axis_index("c")

    def chunk(base, idx_hbm, out_hbm):
      pltpu.sync_copy(idx_hbm.at[pl.ds(base, _GK)], idx_v)
      pltpu.async_copy(node_hbm.at[idx_v], rows_v, sem).wait()
      pltpu.sync_copy(rows_v, out_hbm.at[pl.ds(base, _GK)])

    def loop_body(c, carry):
      base = wid * epw + c * _GK
      chunk(base, src_hbm, xs_hbm)
      chunk(base, dst_hbm, xd_hbm)
      return carry

    lax.fori_loop(0, epw // _GK, loop_body, 0)

  return body(node_f, edge_src, edge_dst)


def _sc_zscatter(expT, edge_dst):
  ne = edge_dst.shape[0]
  epw = ne // NW
  W = (epw // 128 + 2) * 128
  mesh = plsc.VectorSubcoreMesh(core_axis_name="c", subcore_axis_name="s")

  @functools.partial(
      pl.kernel,
      out_type=jax.ShapeDtypeStruct((NW, H, N_PAD), jnp.float32),
      mesh=mesh,
      compiler_params=pltpu.CompilerParams(needs_layout_passes=False),
      scratch_types=[pltpu.VMEM((H, W), jnp.float32),
                     pltpu.VMEM((epw,), jnp.int32),
                     pltpu.VMEM((H, N_PAD), jnp.float32)],
  )
  def body(exp_hbm, dst_hbm, zp_hbm, exp_v, dst_v, z_v):
    wid = lax.axis_index("s") * NC + lax.axis_index("c")
    start = wid * epw
    base = jnp.minimum((start // 128) * 128, ne - W)
    off = start - base
    pltpu.sync_copy(exp_hbm.at[:, pl.ds(base, W)], exp_v)
    pltpu.sync_copy(dst_hbm.at[pl.ds(start, epw)], dst_v)

    zero = jnp.zeros((LANES,), jnp.float32)

    def zbody(i, carry):
      h = i // (N_PAD // LANES)
      n = i % (N_PAD // LANES)
      z_v[h, pl.ds(n * LANES, LANES)] = zero
      return carry

    lax.fori_loop(0, H * N_PAD // LANES, zbody, 0)

    def sbody(g, carry):
      dstg = dst_v[pl.ds(g * LANES, LANES)]
      for h in range(H):
        vals = exp_v[h, pl.ds(off + g * LANES, LANES)]
        hvec = jnp.full((LANES,), h, jnp.int32)
        plsc.addupdate_scatter(z_v, [hvec, dstg], vals)
      return carry

    lax.fori_loop(0, epw // LANES, sbody, 0)
    pltpu.sync_copy(z_v, zp_hbm.at[wid])

  return body(expT, edge_dst)


_UK = 200
_DRAIN = 80
_NDCHUNK = N_PAD // _DRAIN


def _sc_uscatter(u, edge_dst):
  epw = edge_dst.shape[0] // NW
  mesh = plsc.VectorSubcoreMesh(core_axis_name="c", subcore_axis_name="s")

  @functools.partial(
      pl.kernel,
      out_type=jax.ShapeDtypeStruct((NC, N_PAD, D), jnp.float32),
      mesh=mesh,
      compiler_params=pltpu.CompilerParams(needs_layout_passes=False),
      scratch_types=[pltpu.VMEM((_UK, D), jnp.float32),
                     pltpu.VMEM((_UK,), jnp.int32),
                     pltpu.VMEM_SHARED((N_PAD, D), jnp.float32)],
  )
  def body(u_hbm, dst_hbm, out_hbm, u_v, dst_v, acc_sh):
    cid = lax.axis_index("c")
    sid = lax.axis_index("s")
    wid = sid * NC + cid

    zero = jnp.zeros((LANES,), jnp.float32)

    def zfill(i, carry):
      u_v[i // (D // LANES), pl.ds((i % (D // LANES)) * LANES, LANES)] = zero
      return carry

    lax.fori_loop(0, _DRAIN * D // LANES, zfill, 0)

    def zcopy(j, carry):
      c = sid + j * NS

      @pl.when(c < _NDCHUNK)
      def _():
        pltpu.sync_copy(u_v.at[pl.ds(0, _DRAIN)], acc_sh.at[pl.ds(c * _DRAIN, _DRAIN)])

      return carry

    lax.fori_loop(0, (_NDCHUNK + NS - 1) // NS, zcopy, 0)
    plsc.subcore_barrier()

    def cbody(k, carry):
      base = wid * epw + k * _UK
      pltpu.sync_copy(dst_hbm.at[pl.ds(base, _UK)], dst_v)
      pltpu.sync_copy(u_hbm.at[pl.ds(base, _UK)], u_v)
      pltpu.sync_copy(u_v, acc_sh.at[dst_v], add=True)
      return carry

    lax.fori_loop(0, epw // _UK, cbody, 0)
    plsc.subcore_barrier()

    def dbody(j, carry):
      c = sid + j * NS

      @pl.when(c < _NDCHUNK)
      def _():
        row = c * _DRAIN
        dr_v = u_v.at[pl.ds(0, _DRAIN)]
        pltpu.sync_copy(acc_sh.at[pl.ds(row, _DRAIN)], dr_v)
        pltpu.sync_copy(dr_v, out_hbm.at[cid, pl.ds(row, _DRAIN)])

      return carry

    lax.fori_loop(0, (_NDCHUNK + NS - 1) // NS, dbody, 0)

  return body(u, edge_dst)


def kernel(edge_src, edge_dst, edge_scalar_attr, edge_attr, edge_weight_cutoff,
           node_f, Wk1, Wk2, Wk3, Wv1, Wv2, Wv3, Wdot, Wlin):
  bf16 = jnp.bfloat16
  Wdot_r = jnp.transpose(Wdot, (1, 2, 0)).reshape(D, H * D).astype(bf16)
  Wk1, Wk2, Wk3 = Wk1.astype(bf16), Wk2.astype(bf16), Wk3.astype(bf16)
  Wv1, Wv2, Wv3 = Wv1.astype(bf16), Wv2.astype(bf16), Wv3.astype(bf16)

  esaT = edge_scalar_attr.T
  eaT = edge_attr.T
  cutT = edge_weight_cutoff[None, :]

  nch = PIPE_CHUNKS
  ce = N_EDGES // nch

  gathered = [_sc_gather(node_f,
                         lax.slice_in_dim(edge_src, c * ce, (c + 1) * ce),
                         lax.slice_in_dim(edge_dst, c * ce, (c + 1) * ce))
              for c in range(nch)]
  exps, us, accs, zparts = [], [], [], []
  for c in range(nch):
    xs, xd = gathered[c]
    expv, u = _tc_edge(lax.slice_in_dim(esaT, c * ce, (c + 1) * ce, axis=1),
                       xs, xd,
                       lax.slice_in_dim(eaT, c * ce, (c + 1) * ce, axis=1),
                       lax.slice_in_dim(cutT, c * ce, (c + 1) * ce, axis=1),
                       Wk1, Wk2, Wk3, Wv1, Wv2, Wv3, Wdot_r)
    exps.append(expv)
    us.append(u)
  for c in range(nch):
    dst_c = lax.slice_in_dim(edge_dst, c * ce, (c + 1) * ce)
    zparts.append(_sc_zscatter(exps[c], dst_c))
    accs.append(_sc_uscatter(us[c], dst_c))
  zpart = zparts[0] if nch == 1 else jnp.concatenate(zparts, axis=0)
  acc = accs[0] if nch == 1 else jnp.concatenate(accs, axis=0)

  return lax.slice_in_dim(_tc_final(acc, zpart, Wlin), 0, N_NODES)
